# baseline (speedup 1.0000x reference)
import functools

import jax
import jax.numpy as jnp
from jax.experimental import pallas as pl
from jax.experimental.pallas import tpu as pltpu

B, S, D_RHO, D_C, HID = 16, 2048, 256, 128, 512
GRID = 17
K_PAD = 384

_HI = jax.lax.Precision.HIGHEST


def _fused_kernel(rho_ref, wcol_ref, idx_ref, cflat_ref, W1_ref, b1_ref,
                  W2_ref, b2_ref, Wf1r_ref, Wf1c_ref, bf1_ref, W2b_ref,
                  outm_ref, outv_ref):
    idx_col = idx_ref[0]
    bins = jax.lax.broadcasted_iota(jnp.int32, (S, K_PAD), 1)
    onehot = (bins == idx_col).astype(jnp.float32)
    hist = jnp.sum(onehot, axis=0, keepdims=True)
    mean_c = jnp.dot(hist * (1.0 / S), cflat_ref[...], precision=_HI)

    pre = jnp.dot(mean_c, W1_ref[...], precision=_HI) + b1_ref[...]
    scale, alpha = 1.0507009873554805, 1.6732632423543772
    h2 = scale * jnp.where(pre > 0, pre, alpha * (jnp.exp(pre) - 1.0))
    c_tr = jnp.dot(h2, W2_ref[...], precision=_HI) + b2_ref[...]
    bias = jnp.dot(c_tr, Wf1c_ref[...], precision=_HI) + bf1_ref[...]

    acc = jnp.dot(rho_ref[0].astype(jnp.bfloat16), Wf1r_ref[...],
                  preferred_element_type=jnp.float32)
    hf = jnp.maximum(acc + bias, 0.0)
    lmv = jnp.dot(hf.astype(jnp.bfloat16), W2b_ref[...],
                  preferred_element_type=jnp.float32)

    logw = jnp.log(wcol_ref[0])
    a = lmv[:, 0:1] + logw
    bb = lmv[:, 1:2] + 2.0 * logw
    ma = jnp.max(a)
    mb = jnp.max(bb)
    la = ma + jnp.log(jnp.sum(jnp.exp(a - ma)))
    lb = mb + jnp.log(jnp.sum(jnp.exp(bb - mb)))
    outm_ref[0] = jnp.full((8, 128), la, dtype=jnp.float32)
    outv_ref[0] = jnp.full((8, 128), lb, dtype=jnp.float32)


@jax.jit
def kernel(rho, c, w, l, roads, lon_idx, lat_idx, W1, b1, W2, b2, Wf1, bf1,
           W21, b21, W22, b22):
    cc = jnp.transpose(jnp.squeeze(c, axis=0), (1, 2, 0))
    c_flat = cc.reshape(GRID * GRID, D_C)
    c_flat = jnp.pad(c_flat, ((0, K_PAD - GRID * GRID), (0, 0)))

    idx = (lat_idx.astype(jnp.int32) * GRID + lon_idx.astype(jnp.int32))
    idx_col = idx.reshape(B, S, 1)
    w_col = w.reshape(B, S, 1)

    Wf1_r = Wf1[:D_RHO].astype(jnp.bfloat16)
    Wf1_c = Wf1[D_RHO:]
    W2b = jnp.pad(jnp.concatenate([W21, W22], axis=1),
                  ((0, 0), (0, 126))).astype(jnp.bfloat16)

    full = lambda shp: pl.BlockSpec(shp, lambda b: (0,) * len(shp))
    outm, outv = pl.pallas_call(
        _fused_kernel,
        grid=(B,),
        in_specs=[
            pl.BlockSpec((1, S, D_RHO), lambda b: (b, 0, 0)),
            pl.BlockSpec((1, S, 1), lambda b: (b, 0, 0)),
            pl.BlockSpec((1, S, 1), lambda b: (b, 0, 0)),
            full((K_PAD, D_C)),
            full((D_C, 256)),
            full((1, 256)),
            full((256, D_C)),
            full((1, D_C)),
            full((D_RHO, HID)),
            full((D_C, HID)),
            full((1, HID)),
            full((HID, 128)),
        ],
        out_specs=[
            pl.BlockSpec((1, 8, 128), lambda b: (b, 0, 0)),
            pl.BlockSpec((1, 8, 128), lambda b: (b, 0, 0)),
        ],
        out_shape=[
            jax.ShapeDtypeStruct((B, 8, 128), jnp.float32),
            jax.ShapeDtypeStruct((B, 8, 128), jnp.float32),
        ],
        compiler_params=pltpu.CompilerParams(
            dimension_semantics=("parallel",)),
    )(rho, w_col, idx_col, c_flat, W1, b1.reshape(1, 256), W2,
      b2.reshape(1, D_C), Wf1_r, Wf1_c, bf1.reshape(1, HID), W2b)

    logm_agg = outm[:, 0, 0] + b21[0]
    logv_agg = outv[:, 0, 0] + b22[0]
    logl = jnp.log(l)
    return (logl - logm_agg, logl - 3.0 * logm_agg - logv_agg)


# speedup vs baseline: 1.2114x; 1.2114x over previous
import jax
import jax.numpy as jnp
from jax.experimental import pallas as pl
from jax.experimental.pallas import tpu as pltpu

B, S, D_RHO, D_C, HID = 16, 2048, 256, 128, 512
GRID = 17
K_PAD = 384

_HI = jax.lax.Precision.HIGHEST


def _hist_mlp_bias(idx_col, cflat_ref, W1_ref, b1_ref, W2_ref, b2_ref,
                   Wf1c_ref, bf1_ref):
    bins = jax.lax.broadcasted_iota(jnp.int32, (S, K_PAD), 1)
    onehot = (bins == idx_col).astype(jnp.float32)
    hist = jnp.sum(onehot, axis=0, keepdims=True)
    mean_c = jnp.dot(hist * (1.0 / S), cflat_ref[...])
    pre = jnp.dot(mean_c, W1_ref[...]) + b1_ref[...]
    scale, alpha = 1.0507009873554805, 1.6732632423543772
    h2 = scale * jnp.where(pre > 0, pre, alpha * (jnp.exp(pre) - 1.0))
    c_tr = jnp.dot(h2, W2_ref[...]) + b2_ref[...]
    return jnp.dot(c_tr, Wf1c_ref[...]) + bf1_ref[...]


def _tc_body(idxn_ref, idx0_ref, rho_ref, wcol_ref, cflat_ref, W1_ref,
             b1_ref, W2_ref, b2_ref, Wf1r_ref, Wf1c_ref, bf1_ref, W2b_ref,
             outm_ref, outv_ref, bias_sc, lmv_sc):
    b = pl.program_id(0)

    @pl.when(b == 0)
    def _bias0():
        bias_sc[pl.ds(0, 1), :] = _hist_mlp_bias(
            idx0_ref[0], cflat_ref, W1_ref, b1_ref, W2_ref, b2_ref,
            Wf1c_ref, bf1_ref)

    lmp = lmv_sc[pl.ds(1 - b % 2, 1)][0]
    w = wcol_ref[0]
    lm = lmp[:, 0:1]
    lv = lmp[:, 1:2]
    ma = jnp.max(lm)
    mb = jnp.max(lv)
    sa = jnp.sum(w * jnp.exp(lm - ma))
    sb = jnp.sum((w * w) * jnp.exp(lv - mb))
    outm_ref[0] = jnp.full((8, 128), ma + jnp.log(sa), dtype=jnp.float32)
    outv_ref[0] = jnp.full((8, 128), mb + jnp.log(sb), dtype=jnp.float32)

    bias = bias_sc[pl.ds(jnp.minimum(b, B - 1), 1), :]
    acc = jnp.dot(rho_ref[0].astype(jnp.bfloat16), Wf1r_ref[...],
                  preferred_element_type=jnp.float32)
    hf = jnp.maximum(acc + bias, 0.0)
    lmv = jnp.dot(hf.astype(jnp.bfloat16), W2b_ref[...],
                  preferred_element_type=jnp.float32)
    lmv_sc[pl.ds(b % 2, 1)] = lmv[None]

    bias_sc[pl.ds(jnp.minimum(b + 1, B - 1), 1), :] = _hist_mlp_bias(
        idxn_ref[0], cflat_ref, W1_ref, b1_ref, W2_ref, b2_ref,
        Wf1c_ref, bf1_ref)


@jax.jit
def kernel(rho, c, w, l, roads, lon_idx, lat_idx, W1, b1, W2, b2, Wf1, bf1,
           W21, b21, W22, b22):
    cc = jnp.transpose(jnp.squeeze(c, axis=0), (1, 2, 0))
    c_flat = cc.reshape(GRID * GRID, D_C)
    c_flat = jnp.pad(c_flat, ((0, K_PAD - GRID * GRID), (0, 0)))

    idx = (lat_idx.astype(jnp.int32) * GRID + lon_idx.astype(jnp.int32))
    idx_col = idx.reshape(B, S, 1)
    w_col = w.reshape(B, S, 1)

    Wf1_r = Wf1[:D_RHO].astype(jnp.bfloat16)
    Wf1_c = Wf1[D_RHO:]
    W2b = jnp.pad(jnp.concatenate([W21, W22], axis=1),
                  ((0, 0), (0, 126))).astype(jnp.bfloat16)

    full = lambda shp: pl.BlockSpec(shp, lambda b: (0,) * len(shp))
    outm, outv = pl.pallas_call(
        _tc_body,
        grid=(B + 1,),
        in_specs=[
            pl.BlockSpec((1, S, 1), lambda b: (jnp.minimum(b + 1, B - 1), 0, 0)),
            pl.BlockSpec((1, S, 1), lambda b: (0, 0, 0)),
            pl.BlockSpec((1, S, D_RHO), lambda b: (jnp.minimum(b, B - 1), 0, 0)),
            pl.BlockSpec((1, S, 1), lambda b: (jnp.maximum(b - 1, 0), 0, 0)),
            full((K_PAD, D_C)),
            full((D_C, 256)),
            full((1, 256)),
            full((256, D_C)),
            full((1, D_C)),
            full((D_RHO, HID)),
            full((D_C, HID)),
            full((1, HID)),
            full((HID, 128)),
        ],
        out_specs=[
            pl.BlockSpec((1, 8, 128), lambda b: (jnp.maximum(b - 1, 0), 0, 0)),
            pl.BlockSpec((1, 8, 128), lambda b: (jnp.maximum(b - 1, 0), 0, 0)),
        ],
        out_shape=[
            jax.ShapeDtypeStruct((B, 8, 128), jnp.float32),
            jax.ShapeDtypeStruct((B, 8, 128), jnp.float32),
        ],
        scratch_shapes=[
            pltpu.VMEM((B, HID), jnp.float32),
            pltpu.VMEM((2, S, 128), jnp.float32),
        ],
        compiler_params=pltpu.CompilerParams(
            dimension_semantics=("arbitrary",)),
    )(idx_col, idx_col, rho, w_col, c_flat, W1, b1.reshape(1, 256), W2,
      b2.reshape(1, D_C), Wf1_r, Wf1_c, bf1.reshape(1, HID), W2b)

    logm_agg = outm[:, 0, 0] + b21[0]
    logv_agg = outv[:, 0, 0] + b22[0]
    logl = jnp.log(l)
    return (logl - logm_agg, logl - 3.0 * logm_agg - logv_agg)


# speedup vs baseline: 1.2393x; 1.0230x over previous
import jax
import jax.numpy as jnp
from jax.experimental import pallas as pl
from jax.experimental.pallas import tpu as pltpu

B, S, D_RHO, D_C, HID = 16, 2048, 256, 128, 512
GRID = 17
K_PAD = 384

_HI = jax.lax.Precision.HIGHEST


def _hist(idx_col):
    bins = jax.lax.broadcasted_iota(jnp.int32, (S, K_PAD), 1)
    onehot = (bins == idx_col).astype(jnp.float32)
    return jnp.sum(onehot, axis=0, keepdims=True)


def _mlp_bias(hist, cflat_ref, W1_ref, b1_ref, W2_ref, b2_ref,
              Wf1c_ref, bf1_ref):
    mean_c = jnp.dot(hist * (1.0 / S), cflat_ref[...])
    pre = jnp.dot(mean_c, W1_ref[...]) + b1_ref[...]
    scale, alpha = 1.0507009873554805, 1.6732632423543772
    h2 = scale * jnp.where(pre > 0, pre, alpha * (jnp.exp(pre) - 1.0))
    c_tr = jnp.dot(h2, W2_ref[...]) + b2_ref[...]
    return jnp.dot(c_tr, Wf1c_ref[...]) + bf1_ref[...]


def _tc_body(idxn_ref, idx0_ref, idx1_ref, rho_ref, wcol_ref, cflat_ref,
             W1_ref, b1_ref, W2_ref, b2_ref, Wf1r_ref, Wf1c_ref, bf1_ref,
             W2b_ref, outm_ref, outv_ref, bias_sc, lmv_sc, hist_sc):
    b = pl.program_id(0)

    @pl.when(b == 0)
    def _prologue():
        bias_sc[pl.ds(0, 1), :] = _mlp_bias(
            _hist(idx0_ref[0]), cflat_ref, W1_ref, b1_ref, W2_ref, b2_ref,
            Wf1c_ref, bf1_ref)
        hist_sc[pl.ds(1, 1)] = _hist(idx1_ref[0])[None]

    lmp = lmv_sc[pl.ds(1 - b % 2, 1)][0]
    w = wcol_ref[0]
    lm = lmp[:, 0:1]
    lv = lmp[:, 1:2]
    ma = jnp.max(lm)
    mb = jnp.max(lv)
    sa = jnp.sum(w * jnp.exp(lm - ma))
    sb = jnp.sum((w * w) * jnp.exp(lv - mb))
    outm_ref[0] = jnp.full((8, 128), ma + jnp.log(sa), dtype=jnp.float32)
    outv_ref[0] = jnp.full((8, 128), mb + jnp.log(sb), dtype=jnp.float32)

    bias = bias_sc[pl.ds(jnp.minimum(b, B - 1), 1), :]
    acc = jnp.dot(rho_ref[0].astype(jnp.bfloat16), Wf1r_ref[...],
                  preferred_element_type=jnp.float32)
    hf = jnp.maximum(acc + bias, 0.0).astype(jnp.bfloat16)
    lmv = jnp.dot(hf, W2b_ref[...],
                  preferred_element_type=jnp.float32)
    lmv_sc[pl.ds(b % 2, 1)] = lmv[None]

    histp = hist_sc[pl.ds(1 - b % 2, 1)][0]
    bias_sc[pl.ds(jnp.minimum(b + 1, B - 1), 1), :] = _mlp_bias(
        histp, cflat_ref, W1_ref, b1_ref, W2_ref, b2_ref, Wf1c_ref, bf1_ref)

    hist_sc[pl.ds(b % 2, 1)] = _hist(idxn_ref[0])[None]


@jax.jit
def kernel(rho, c, w, l, roads, lon_idx, lat_idx, W1, b1, W2, b2, Wf1, bf1,
           W21, b21, W22, b22):
    cc = jnp.transpose(jnp.squeeze(c, axis=0), (1, 2, 0))
    c_flat = cc.reshape(GRID * GRID, D_C)
    c_flat = jnp.pad(c_flat, ((0, K_PAD - GRID * GRID), (0, 0)))

    idx = (lat_idx.astype(jnp.int32) * GRID + lon_idx.astype(jnp.int32))
    idx_col = idx.reshape(B, S, 1)
    w_col = w.reshape(B, S, 1)

    Wf1_r = Wf1[:D_RHO].astype(jnp.bfloat16)
    Wf1_c = Wf1[D_RHO:]
    W2b = jnp.pad(jnp.concatenate([W21, W22], axis=1),
                  ((0, 0), (0, 126))).astype(jnp.bfloat16)

    full = lambda shp: pl.BlockSpec(shp, lambda b: (0,) * len(shp))
    outm, outv = pl.pallas_call(
        _tc_body,
        grid=(B + 1,),
        in_specs=[
            pl.BlockSpec((1, S, 1), lambda b: (jnp.minimum(b + 2, B - 1), 0, 0)),
            pl.BlockSpec((1, S, 1), lambda b: (0, 0, 0)),
            pl.BlockSpec((1, S, 1), lambda b: (1, 0, 0)),
            pl.BlockSpec((1, S, D_RHO), lambda b: (jnp.minimum(b, B - 1), 0, 0)),
            pl.BlockSpec((1, S, 1), lambda b: (jnp.maximum(b - 1, 0), 0, 0)),
            full((K_PAD, D_C)),
            full((D_C, 256)),
            full((1, 256)),
            full((256, D_C)),
            full((1, D_C)),
            full((D_RHO, HID)),
            full((D_C, HID)),
            full((1, HID)),
            full((HID, 128)),
        ],
        out_specs=[
            pl.BlockSpec((1, 8, 128), lambda b: (jnp.maximum(b - 1, 0), 0, 0)),
            pl.BlockSpec((1, 8, 128), lambda b: (jnp.maximum(b - 1, 0), 0, 0)),
        ],
        out_shape=[
            jax.ShapeDtypeStruct((B, 8, 128), jnp.float32),
            jax.ShapeDtypeStruct((B, 8, 128), jnp.float32),
        ],
        scratch_shapes=[
            pltpu.VMEM((B, HID), jnp.float32),
            pltpu.VMEM((2, S, 128), jnp.float32),
            pltpu.VMEM((2, 1, K_PAD), jnp.float32),
        ],
        compiler_params=pltpu.CompilerParams(
            dimension_semantics=("arbitrary",)),
    )(idx_col, idx_col, idx_col, rho, w_col, c_flat, W1, b1.reshape(1, 256),
      W2, b2.reshape(1, D_C), Wf1_r, Wf1_c, bf1.reshape(1, HID), W2b)

    logm_agg = outm[:, 0, 0] + b21[0]
    logv_agg = outv[:, 0, 0] + b22[0]
    logl = jnp.log(l)
    return (logl - logm_agg, logl - 3.0 * logm_agg - logv_agg)
